# Initial kernel scaffold; baseline (speedup 1.0000x reference)
import jax
import jax.numpy as jnp
from jax.experimental import pallas as pl


def kernel(user, item, theta, prompt_theta, a, s_vecs, b_table, W1, b1, W2, b2):
    raise NotImplementedError("write your pallas kernel here")


# speedup vs baseline: 1.2185x; 1.2185x over previous
import functools

import jax
import jax.numpy as jnp
from jax import lax
from jax.experimental import pallas as pl
from jax.experimental.pallas import tpu as pltpu
from jax.experimental.pallas import tpu_sc as plsc

_NC = 2
_NS = 16
_NW = _NC * _NS
_LANES = 16


def _sc_gather_body(U, BPW,
                    user_hbm, item_hbm, theta_hbm, prompt_hbm, a_hbm, bt_hbm,
                    out_p, out_t, out_a, out_b,
                    uidx, pidx, iidx, pbuf, tbuf, abuf, bbuf,
                    s0, s1, s2, s3):
    wid = lax.axis_index("s") * _NC + lax.axis_index("c")
    base = wid * BPW
    pltpu.sync_copy(user_hbm.at[pl.ds(base, BPW)], uidx)
    pltpu.sync_copy(item_hbm.at[pl.ds(base, BPW)], iidx)
    c_t = pltpu.async_copy(theta_hbm.at[uidx], tbuf, s0)
    c_a = pltpu.async_copy(a_hbm.at[iidx], abuf, s1)
    c_b = pltpu.async_copy(bt_hbm.at[iidx], bbuf, s2)
    for i in range(BPW // _LANES):
        sl = pl.ds(i * _LANES, _LANES)
        pidx[sl] = lax.rem(uidx[sl], U)
    c_p = pltpu.async_copy(prompt_hbm.at[pidx], pbuf, s3)
    c_t.wait()
    c_a.wait()
    c_b.wait()
    c_p.wait()
    pltpu.sync_copy(pbuf, out_p.at[pl.ds(base, BPW)])
    pltpu.sync_copy(tbuf, out_t.at[pl.ds(base, BPW)])
    pltpu.sync_copy(abuf, out_a.at[pl.ds(base, BPW)])
    pltpu.sync_copy(bbuf, out_b.at[pl.ds(base, BPW)])


@functools.lru_cache(maxsize=None)
def _make_sc_gather(B, U, RU, I, L, P):
    BPW = B // _NW
    f32 = jnp.float32
    mesh = plsc.VectorSubcoreMesh(core_axis_name="c", subcore_axis_name="s")
    return pl.kernel(
        functools.partial(_sc_gather_body, U, BPW),
        mesh=mesh,
        compiler_params=pltpu.CompilerParams(use_tc_tiling_on_sc=False),
        out_type=[
            jax.ShapeDtypeStruct((B, P), f32),
            jax.ShapeDtypeStruct((B, L), f32),
            jax.ShapeDtypeStruct((B, L), f32),
            jax.ShapeDtypeStruct((B, 1), f32),
        ],
        scratch_types=[
            pltpu.VMEM((BPW,), jnp.int32),
            pltpu.VMEM((BPW,), jnp.int32),
            pltpu.VMEM((BPW,), jnp.int32),
            pltpu.VMEM((BPW, P), f32),
            pltpu.VMEM((BPW, L), f32),
            pltpu.VMEM((BPW, L), f32),
            pltpu.VMEM((BPW, 1), f32),
            pltpu.SemaphoreType.DMA,
            pltpu.SemaphoreType.DMA,
            pltpu.SemaphoreType.DMA,
            pltpu.SemaphoreType.DMA,
        ],
    )


def _tc_dense_body(P, split,
                   p_ref, t_ref, a_ref, bb_ref, item_ref,
                   w1_ref, b1_ref, w2_ref, b2_ref, s_ref, o_ref):
    w1 = w1_ref[...]
    w2 = w2_ref[...]
    dn = (((1,), (1,)), ((), ()))
    p = jax.nn.sigmoid(p_ref[...])
    t = jax.nn.sigmoid(t_ref[...])
    nt = lax.dot_general(p, w1[:, :P], dn, preferred_element_type=jnp.float32)
    nt = nt + lax.dot_general(t, w1[:, P:], dn, preferred_element_type=jnp.float32)
    nt = jax.nn.sigmoid(nt + b1_ref[...])
    av = jax.nn.sigmoid(a_ref[...])
    na = lax.dot_general(av, w2[:, P:], dn, preferred_element_type=jnp.float32)
    s_sig = jax.nn.sigmoid(s_ref[...])
    s2 = lax.dot_general(s_sig, w2[:, :P], dn, preferred_element_type=jnp.float32)
    item = item_ref[...]
    R = s2.shape[0]
    s_c = jnp.zeros_like(na)
    for r in range(R):
        in_r = jnp.logical_and(item >= r * split, item < (r + 1) * split)
        s_c = s_c + jnp.where(in_r, s2[r:r + 1, :], 0.0)
    na = jax.nn.sigmoid(na + s_c + b2_ref[...])
    nb = jax.nn.sigmoid(bb_ref[...])
    o_ref[...] = jax.nn.sigmoid(
        jnp.sum(na * nt, axis=-1, keepdims=True) - nb)


def kernel(user, item, theta, prompt_theta, a, s_vecs, b_table, W1, b1, W2, b2):
    B = user.shape[0]
    R, U, L = theta.shape
    P = prompt_theta.shape[1]
    I = a.shape[0]
    split = I // s_vecs.shape[0]

    user_i = user.astype(jnp.int32)
    item_i = item.astype(jnp.int32)
    theta_flat = theta.reshape(R * U, L)

    p_g, t_g, a_g, b_g = _make_sc_gather(B, U, R * U, I, L, P)(
        user_i, item_i, theta_flat, prompt_theta, a, b_table)

    BLK = 2048
    grid = (B // BLK,)
    out = pl.pallas_call(
        functools.partial(_tc_dense_body, P, split),
        grid=grid,
        in_specs=[
            pl.BlockSpec((BLK, P), lambda i: (i, 0)),
            pl.BlockSpec((BLK, L), lambda i: (i, 0)),
            pl.BlockSpec((BLK, L), lambda i: (i, 0)),
            pl.BlockSpec((BLK, 1), lambda i: (i, 0)),
            pl.BlockSpec((BLK, 1), lambda i: (i, 0)),
            pl.BlockSpec(W1.shape, lambda i: (0, 0)),
            pl.BlockSpec((1, L), lambda i: (0, 0)),
            pl.BlockSpec(W2.shape, lambda i: (0, 0)),
            pl.BlockSpec((1, L), lambda i: (0, 0)),
            pl.BlockSpec(s_vecs.shape, lambda i: (0, 0)),
        ],
        out_specs=pl.BlockSpec((BLK, 1), lambda i: (i, 0)),
        out_shape=jax.ShapeDtypeStruct((B, 1), jnp.float32),
    )(p_g, t_g, a_g, b_g, item_i.reshape(B, 1),
      W1, b1.reshape(1, L), W2, b2.reshape(1, L), s_vecs)
    return out.reshape(B)
